# baseline (speedup 1.0000x reference)
import functools

import jax
import jax.numpy as jnp
from jax import lax
from jax.experimental import pallas as pl
from jax.experimental.pallas import tpu as pltpu
from jax.experimental.pallas import tpu_sc as plsc

D_MODEL = 64
SCALE = 8.0

NC = 2
NS = 16
L = 16
NW = NC * NS


def _worker_id():
    return lax.axis_index("s") * NC + lax.axis_index("c")


@functools.lru_cache(maxsize=None)
def _build_transpose(V):
    n_full = V // 128
    tail = V % 128
    u_tec = (n_full + NW - 1) // NW
    RING = 6
    n_t = (u_tec + RING - 1) // RING
    V2P = n_full * 64 + tail // 2
    mesh = plsc.VectorSubcoreMesh(core_axis_name="c", subcore_axis_name="s")

    @functools.partial(
        pl.kernel,
        mesh=mesh,
        out_type=jax.ShapeDtypeStruct((V2P, 128), jnp.float32),
        scratch_types=(
            [pltpu.VMEM((D_MODEL, 128), jnp.float32) for _ in range(6)]
            + [pltpu.VMEM((D_MODEL, 128), jnp.float32) for _ in range(6)]
            + [pltpu.SemaphoreType.DMA] * 12
        ),
        compiler_params=pltpu.CompilerParams(needs_layout_passes=False),
    )
    def kb(tabT_hbm, tail_hbm, out_hbm, *bufs):
        srcs = bufs[0:6]
        dsts = bufs[6:12]
        srs = bufs[12:18]
        sws = bufs[18:24]
        wid = _worker_id()
        base = wid * u_tec
        lim = jnp.minimum(base + u_tec, n_full)

        def rd(u, uu):
            return pltpu.make_async_copy(
                tabT_hbm.at[:, pl.ds(u * 128, 128)], srcs[uu], srs[uu])

        def wr_drain(uu):
            pltpu.make_async_copy(
                dsts[uu], out_hbm.at[pl.ds(0, D_MODEL), :], sws[uu]).wait()

        for uu in range(RING):
            @pl.when(base + uu < lim)
            def _(uu=uu):
                rd(base + uu, uu).start()

        def step(t, carry):
            u0 = base + RING * t
            for uu in range(RING):
                u = u0 + uu

                @pl.when((t > 0) & (u - RING < lim))
                def _(uu=uu):
                    wr_drain(uu)

                @pl.when(u < lim)
                def _(u=u, uu=uu):
                    rd(u, uu).wait()
                    src = srcs[uu]
                    dst = dsts[uu]
                    lane = lax.iota(jnp.int32, L)
                    hvec = lax.bitwise_and(lane, 1)
                    jbase = lax.shift_right_logical(lane, 1)
                    for d0 in range(0, D_MODEL, L):
                        @plsc.parallel_loop(0, L, unroll=2)
                        def _(k, d0=d0):
                            perm = lax.bitwise_and(lane + k, 15)
                            jvec = lax.bitwise_and(jbase + k, 7)
                            dvec = perm + d0
                            dcol = hvec * 64 + dvec
                            vvb = 2 * jvec + hvec
                            for r0 in range(0, D_MODEL, 8):
                                vv = vvb + (2 * r0)
                                vals = plsc.load_gather(src, [dvec, vv])
                                plsc.store_scatter(
                                    dst, [jvec + r0, dcol], vals)

                    pltpu.async_copy(
                        dst, out_hbm.at[pl.ds(u * D_MODEL, D_MODEL), :],
                        sws[uu])

                @pl.when(u + RING < lim)
                def _(u=u, uu=uu):
                    rd(u + RING, uu).start()
            return carry

        lax.fori_loop(0, n_t, step, 0)
        for uu in range(RING):
            @pl.when(base + RING * (n_t - 1) + uu < lim)
            def _(uu=uu):
                wr_drain(uu)
        if tail:
            @pl.when(wid == 0)
            def _():
                pltpu.sync_copy(
                    tail_hbm,
                    out_hbm.at[pl.ds(n_full * 64, tail // 2), :])

    return kb


@functools.lru_cache(maxsize=None)
def _build_gather(S, T, V2P):
    n_sj = T // 8
    n_ci = S // 128
    n_units = n_sj * n_ci
    u_per = n_units // NW
    assert n_units % NW == 0 and T % 8 == 0 and S % 128 == 0
    assert n_ci == 32

    mesh = plsc.VectorSubcoreMesh(core_axis_name="c", subcore_axis_name="s")

    @functools.partial(
        pl.kernel,
        mesh=mesh,
        out_type=jax.ShapeDtypeStruct((T, D_MODEL, S), jnp.float32),
        scratch_types=(
            [pltpu.VMEM((8, 128), jnp.int32) for _ in range(4)]
            + [pltpu.VMEM((128, 128), jnp.float32) for _ in range(4)]
            + [pltpu.VMEM((D_MODEL, 128), jnp.float32) for _ in range(2)]
            + [pltpu.SemaphoreType.DMA] * 8
        ),
        compiler_params=pltpu.CompilerParams(needs_layout_passes=False),
    )
    def ka(tab_hbm, tok_hbm, out_hbm, *bufs):
        toks = bufs[0:2]
        idxs = bufs[2:4]
        rows = bufs[4:8]
        dsts = bufs[8:10]
        sgs = bufs[10:14]
        sws = bufs[14:16]
        sts = bufs[16:18]
        wid = _worker_id()
        lane = lax.iota(jnp.int32, L)

        def coords(un):
            sj = lax.shift_right_logical(un, 5)
            ci = lax.bitwise_and(un, 31)
            return sj * 8, ci * 128

        def tok_copy(un, p):
            j0, i0 = coords(un)
            return pltpu.make_async_copy(
                tok_hbm.at[pl.ds(j0, 8), pl.ds(i0, 128)], toks[p], sts[p])

        def gather(un_idx, jj, p):
            return pltpu.make_async_copy(
                tab_hbm.at[idxs[p].at[jj]], rows[jj % 4], sgs[jj % 4])

        def idx_compute(p):
            @plsc.parallel_loop(0, 64, unroll=2)
            def _(s):
                jj = lax.shift_right_logical(s, 3)
                sl = pl.ds(lax.bitwise_and(s, 7) * L, L)
                idxs[p][jj, sl] = lax.shift_right_logical(
                    toks[p][jj, sl], 1)

        tok_copy(wid * u_per, 0).start()

        n_pairs = (u_per + 2) // 2

        def pair(t, carry):
            for p in range(2):
                u = 2 * t + p
                un = wid * u_per + u
                un_next = un + 1
                nxt = (u + 1 < 2 * n_pairs) & (un_next < n_units)

                @pl.when(un < n_units)
                def _(u=u, p=p, un=un, un_next=un_next, nxt=nxt):
                    j0, i0 = coords(un)
                    if p == 0:
                        @pl.when(u == 0)
                        def _():
                            tok_copy(un, p).wait()
                            idx_compute(p)
                            for jj in range(3):
                                gather(un, jj, p).start()

                    @pl.when(nxt)
                    def _():
                        tok_copy(un_next, 1 - p).start()

                    for jj in range(8):
                        if jj + 3 < 8:
                            gather(un, jj + 3, p).start()
                        else:
                            jj2 = jj + 3 - 8

                            @pl.when(nxt)
                            def _(jj2=jj2):
                                if jj2 == 0:
                                    tok_copy(un_next, 1 - p).wait()
                                    idx_compute(1 - p)
                                gather(un_next, jj2, 1 - p).start()
                        b2 = jj % 2

                        def drain():
                            pltpu.make_async_copy(
                                dsts[b2],
                                out_hbm.at[j0, slice(None), pl.ds(i0, 128)],
                                sws[b2],
                            ).wait()
                        if jj >= 2:
                            drain()
                        else:
                            @pl.when(un > wid * u_per)
                            def _():
                                drain()
                        gather(un, jj, p).wait()
                        src = rows[jj % 4]
                        dst = dsts[b2]

                        @plsc.parallel_loop(0, 128, unroll=2)
                        def _(s, jj=jj, src=src, dst=dst, p=p):
                            cb = lax.bitwise_and(s, 112)
                            k = lax.bitwise_and(s, 15)
                            iv = lane + cb
                            colbase = lax.shift_left(
                                lax.bitwise_and(
                                    toks[p][jj, pl.ds(cb, L)], 1), 6)
                            perm = lax.bitwise_and(lane + k, 15)
                            for d0 in range(0, D_MODEL, L):
                                dvec = perm + d0
                                vals = plsc.load_gather(
                                    src, [iv, colbase + dvec])
                                plsc.store_scatter(
                                    dst, [dvec, iv], vals * SCALE)

                        pltpu.async_copy(
                            dst,
                            out_hbm.at[j0 + jj, slice(None), pl.ds(i0, 128)],
                            sws[b2])
            return carry

        lax.fori_loop(0, (u_per + 2) // 2, pair, 0)
        for b in range(2):
            pltpu.make_async_copy(
                dsts[b], out_hbm.at[0, slice(None), pl.ds(0, 128)], sws[b]
            ).wait()

    return ka


@jax.jit
def kernel(tokens, table):
    S, T = tokens.shape
    V, D = table.shape
    tok_t = tokens.T.astype(jnp.int32)
    tabT = table.T
    tail = V % 128
    if tail:
        tail_rows = table[V - tail:].reshape(tail // 2, 2 * D)
    else:
        tail_rows = jnp.zeros((0, 2 * D), table.dtype)
    tab2 = _build_transpose(V)(tabT, tail_rows)
    out_t = _build_gather(S, T, tab2.shape[0])(tab2, tok_t)
    return out_t.transpose(2, 0, 1)


# speedup vs baseline: 1.0454x; 1.0454x over previous
import functools

import jax
import jax.numpy as jnp
from jax import lax
from jax.experimental import pallas as pl
from jax.experimental.pallas import tpu as pltpu
from jax.experimental.pallas import tpu_sc as plsc

D_MODEL = 64
SCALE = 8.0

NC = 2
NS = 16
L = 16
NW = NC * NS


def _worker_id():
    return lax.axis_index("s") * NC + lax.axis_index("c")


@functools.lru_cache(maxsize=None)
def _build_transpose(V):
    n_full = V // 128
    tail = V % 128
    u_tec = (n_full + NW - 1) // NW
    RING = 4
    n_t = (u_tec + RING - 1) // RING
    V2P = n_full * 64 + tail // 2
    mesh = plsc.VectorSubcoreMesh(core_axis_name="c", subcore_axis_name="s")

    @functools.partial(
        pl.kernel,
        mesh=mesh,
        out_type=jax.ShapeDtypeStruct((V2P, 128), jnp.float32),
        scratch_types=(
            [pltpu.VMEM((D_MODEL, 128), jnp.float32) for _ in range(4)]
            + [pltpu.VMEM((D_MODEL, 128), jnp.float32) for _ in range(4)]
            + [pltpu.SemaphoreType.DMA] * 8
        ),
        compiler_params=pltpu.CompilerParams(needs_layout_passes=False),
    )
    def kb(tabT_hbm, tail_hbm, out_hbm, *bufs):
        srcs = bufs[0:4]
        dsts = bufs[4:8]
        srs = bufs[8:12]
        sws = bufs[12:16]
        wid = _worker_id()
        base = wid * u_tec
        lim = jnp.minimum(base + u_tec, n_full)

        def rd(u, uu):
            return pltpu.make_async_copy(
                tabT_hbm.at[:, pl.ds(u * 128, 128)], srcs[uu], srs[uu])

        def wr_drain(uu):
            pltpu.make_async_copy(
                dsts[uu], out_hbm.at[pl.ds(0, D_MODEL), :], sws[uu]).wait()

        for uu in range(RING):
            @pl.when(base + uu < lim)
            def _(uu=uu):
                rd(base + uu, uu).start()

        def step(t, carry):
            u0 = base + RING * t
            for uu in range(RING):
                u = u0 + uu

                @pl.when((t > 0) & (u - RING < lim))
                def _(uu=uu):
                    wr_drain(uu)

                @pl.when(u < lim)
                def _(u=u, uu=uu):
                    rd(u, uu).wait()
                    src = srcs[uu]
                    dst = dsts[uu]
                    lane = lax.iota(jnp.int32, L)
                    hvec = lax.bitwise_and(lane, 1)
                    jbase = lax.shift_right_logical(lane, 1)
                    for d0 in range(0, D_MODEL, L):
                        @plsc.parallel_loop(0, L, unroll=2)
                        def _(k, d0=d0):
                            perm = lax.bitwise_and(lane + k, 15)
                            jvec = lax.bitwise_and(jbase + k, 7)
                            dvec = perm + d0
                            dcol = hvec * 64 + dvec
                            vvb = 2 * jvec + hvec
                            for r0 in range(0, D_MODEL, 8):
                                vv = vvb + (2 * r0)
                                vals = plsc.load_gather(src, [dvec, vv])
                                plsc.store_scatter(
                                    dst, [jvec + r0, dcol], vals)

                    pltpu.async_copy(
                        dst, out_hbm.at[pl.ds(u * D_MODEL, D_MODEL), :],
                        sws[uu])

                @pl.when(u + RING < lim)
                def _(u=u, uu=uu):
                    rd(u + RING, uu).start()
            return carry

        lax.fori_loop(0, n_t, step, 0)
        for uu in range(RING):
            @pl.when(base + RING * (n_t - 1) + uu < lim)
            def _(uu=uu):
                wr_drain(uu)
        if tail:
            @pl.when(wid == 0)
            def _():
                pltpu.sync_copy(
                    tail_hbm,
                    out_hbm.at[pl.ds(n_full * 64, tail // 2), :])

    return kb


@functools.lru_cache(maxsize=None)
def _build_gather(S, T, V2P):
    n_sj = T // 8
    n_ci = S // 128
    n_units = n_sj * n_ci
    u_per = n_units // NW
    assert n_units % NW == 0 and T % 8 == 0 and S % 128 == 0
    assert n_ci == 32

    mesh = plsc.VectorSubcoreMesh(core_axis_name="c", subcore_axis_name="s")

    @functools.partial(
        pl.kernel,
        mesh=mesh,
        out_type=jax.ShapeDtypeStruct((T, D_MODEL, S), jnp.float32),
        scratch_types=(
            [pltpu.VMEM((8, 128), jnp.int32) for _ in range(4)]
            + [pltpu.VMEM((128, 128), jnp.float32) for _ in range(4)]
            + [pltpu.VMEM((D_MODEL, 128), jnp.float32) for _ in range(2)]
            + [pltpu.SemaphoreType.DMA] * 8
        ),
        compiler_params=pltpu.CompilerParams(needs_layout_passes=False),
    )
    def ka(tab_hbm, tok_hbm, out_hbm, *bufs):
        toks = bufs[0:2]
        idxs = bufs[2:4]
        rows = bufs[4:8]
        dsts = bufs[8:10]
        sgs = bufs[10:14]
        sws = bufs[14:16]
        sts = bufs[16:18]
        wid = _worker_id()
        lane = lax.iota(jnp.int32, L)

        def coords(un):
            sj = lax.shift_right_logical(un, 5)
            ci = lax.bitwise_and(un, 31)
            return sj * 8, ci * 128

        def tok_copy(un, p):
            j0, i0 = coords(un)
            return pltpu.make_async_copy(
                tok_hbm.at[pl.ds(j0, 8), pl.ds(i0, 128)], toks[p], sts[p])

        def gather(un_idx, jj, p):
            return pltpu.make_async_copy(
                tab_hbm.at[idxs[p].at[jj]], rows[jj % 4], sgs[jj % 4])

        def idx_compute(p):
            @plsc.parallel_loop(0, 64, unroll=2)
            def _(s):
                jj = lax.shift_right_logical(s, 3)
                sl = pl.ds(lax.bitwise_and(s, 7) * L, L)
                idxs[p][jj, sl] = lax.shift_right_logical(
                    toks[p][jj, sl], 1)

        tok_copy(wid * u_per, 0).start()

        n_pairs = (u_per + 2) // 2

        def pair(t, carry):
            for p in range(2):
                u = 2 * t + p
                un = wid * u_per + u
                un_next = un + 1
                nxt = (u + 1 < 2 * n_pairs) & (un_next < n_units)

                @pl.when(un < n_units)
                def _(u=u, p=p, un=un, un_next=un_next, nxt=nxt):
                    j0, i0 = coords(un)
                    if p == 0:
                        @pl.when(u == 0)
                        def _():
                            tok_copy(un, p).wait()
                            idx_compute(p)
                            for jj in range(3):
                                gather(un, jj, p).start()

                    @pl.when(nxt)
                    def _():
                        tok_copy(un_next, 1 - p).start()

                    for jj in range(8):
                        if jj + 3 < 8:
                            gather(un, jj + 3, p).start()
                        else:
                            jj2 = jj + 3 - 8

                            @pl.when(nxt)
                            def _(jj2=jj2):
                                if jj2 == 0:
                                    tok_copy(un_next, 1 - p).wait()
                                    idx_compute(1 - p)
                                gather(un_next, jj2, 1 - p).start()
                        b2 = jj % 2

                        def drain():
                            pltpu.make_async_copy(
                                dsts[b2],
                                out_hbm.at[j0, slice(None), pl.ds(i0, 128)],
                                sws[b2],
                            ).wait()
                        if jj >= 2:
                            drain()
                        else:
                            @pl.when(un > wid * u_per)
                            def _():
                                drain()
                        gather(un, jj, p).wait()
                        src = rows[jj % 4]
                        dst = dsts[b2]

                        @plsc.parallel_loop(0, 128, unroll=2)
                        def _(s, jj=jj, src=src, dst=dst, p=p):
                            cb = lax.bitwise_and(s, 112)
                            k = lax.bitwise_and(s, 15)
                            iv = lane + cb
                            colbase = lax.shift_left(
                                lax.bitwise_and(
                                    toks[p][jj, pl.ds(cb, L)], 1), 6)
                            perm = lax.bitwise_and(lane + k, 15)
                            for d0 in range(0, D_MODEL, L):
                                dvec = perm + d0
                                vals = plsc.load_gather(
                                    src, [iv, colbase + dvec])
                                plsc.store_scatter(
                                    dst, [dvec, iv], vals * SCALE)

                        pltpu.async_copy(
                            dst,
                            out_hbm.at[j0 + jj, slice(None), pl.ds(i0, 128)],
                            sws[b2])
            return carry

        lax.fori_loop(0, (u_per + 2) // 2, pair, 0)
        for b in range(2):
            pltpu.make_async_copy(
                dsts[b], out_hbm.at[0, slice(None), pl.ds(0, 128)], sws[b]
            ).wait()

    return ka


@jax.jit
def kernel(tokens, table):
    S, T = tokens.shape
    V, D = table.shape
    tok_t = tokens.T.astype(jnp.int32)
    tabT = table.T
    tail = V % 128
    if tail:
        tail_rows = table[V - tail:].reshape(tail // 2, 2 * D)
    else:
        tail_rows = jnp.zeros((0, 2 * D), table.dtype)
    tab2 = _build_transpose(V)(tabT, tail_rows)
    out_t = _build_gather(S, T, tab2.shape[0])(tab2, tok_t)
    return out_t.transpose(2, 0, 1)
